# baseline (speedup 1.0000x reference)
import jax
import jax.numpy as jnp
from jax import lax
from jax.experimental import pallas as pl
from jax.experimental.pallas import tpu as pltpu
from jax.experimental.pallas import tpu_sc as plsc

FEATURE_SIZE = 100000
FIELD_SIZE = 26
BATCH = 4096
EMB = 64
MAX_LEN = 20
ROWS = FIELD_SIZE * BATCH

NC, NS, L = 2, 16, 16
NW = NC * NS
BSLAB = BATCH // NW
CH = 32
NCHUNK = BSLAB // CH
NT = FIELD_SIZE * NCHUNK
GI = CH * MAX_LEN
NG = GI // 128
NQ = EMB // L
NGRP = GI // L
WR = FIELD_SIZE * BSLAB
TRASH = CH


def _sc_body(idx_h, val_h, len_h, tab_h, out_h,
             idxA, idxB, icA, icB, dxA, dxB, gA, gB, valA, valB,
             lenall, offA, offB, s1A, s1B, plA, plB, plloc, zbuf, S1, S2,
             semg0, semg1, semi0, semi1, semv0, semv1, sems0, sems1):
    sid = lax.axis_index("s")
    wid = sid * NC + lax.axis_index("c")
    pbase = sid * (CH + 1)

    idxs = (idxA, idxB)
    ics = (icA, icB)
    dxs = (dxA, dxB)
    gs = (gA, gB)
    vals = (valA, valB)
    offs = (offA, offB)
    s1s = (s1A, s1B)
    pls = (plA, plB)
    semg = (semg0, semg1)
    semi = (semi0, semi1)
    semv = (semv0, semv1)
    semsc = (sems0, sems1)

    iota16 = lax.iota(jnp.int32, L)
    l_of, b_of = [], []
    for g in range(NGRP):
        r0g = (g * L) // MAX_LEN
        th = (r0g + 1) * MAX_LEN - g * L
        bump = (iota16 >= th).astype(jnp.int32)
        b_of.append(bump)
        l_of.append(iota16 + jnp.int32(g * L - r0g * MAX_LEN)
                    - bump * MAX_LEN)

    def row0(t):
        f = t // NCHUNK
        c = t % NCHUNK
        return f * BATCH + wid * BSLAB + c * CH

    def idx_src(t):
        off = pl.multiple_of(row0(t) * MAX_LEN, 128)
        return idx_h.at[pl.ds(off, GI)]

    def val_src(t):
        off = pl.multiple_of(row0(t) * MAX_LEN, 128)
        return val_h.at[pl.ds(off, GI)]

    def issue_idx(t, p):
        pltpu.async_copy(idx_src(t), idxs[p], semi[p])

    def wait_idx(t, p):
        pltpu.make_async_copy(idx_src(t), idxs[p], semi[p]).wait()

    def issue_val(t, p):
        pltpu.async_copy(val_src(t), vals[p], semv[p])

    def wait_val(t, p):
        pltpu.make_async_copy(val_src(t), vals[p], semv[p]).wait()

    def issue_gathers(p, nc, live):
        for j in range(NG):
            @pl.when(jnp.logical_and(live, jnp.int32(j * 128) < nc))
            def _():
                pltpu.async_copy(
                    tab_h.at[ics[p].at[pl.ds(j * 128, 128)]],
                    gs[p].at[pl.ds(j * 128, 128)], semg[p])

    def wait_gathers(p, nc):
        for j in range(NG):
            @pl.when(jnp.int32(j * 128) < nc)
            def _():
                pltpu.make_async_copy(
                    tab_h.at[ics[p].at[pl.ds(j * 128, 128)]],
                    gs[p].at[pl.ds(j * 128, 128)], semg[p]).wait()

    def issue_scatter(p):
        pltpu.async_copy(gs[p], pls[p].at[dxs[p]], semsc[p], add=True)

    def wait_scatter(p):
        pltpu.make_async_copy(gs[p], pls[p].at[dxs[p]], semsc[p]).wait()

    def compact(t, p):
        lbase = t * CH
        la = lenall[pl.ds(lbase, L)]
        lb = lenall[pl.ds(lbase + L, L)]
        offa = plsc.cumsum(la) - la
        offb = plsc.cumsum(lb) - lb
        tota = jnp.sum(la)
        offs[p][pl.ds(0, L)] = offa
        offs[p][pl.ds(L, L)] = offb + jnp.full((L,), tota, jnp.int32)
        nc = tota + jnp.sum(lb)
        trash16 = jnp.full((L,), pbase + CH, jnp.int32)
        row_off = jnp.full((L,), pbase, jnp.int32)
        for g in range(NGRP):
            dxs[p][pl.ds(g * L, L)] = trash16
        for g in range(NGRP):
            idx16 = idxs[p][pl.ds(g * L, L)]
            r0 = (g * L) // MAX_LEN
            l0 = (g * L) % MAX_LEN
            ov = offs[p][pl.ds(r0, L)]
            lv0 = lenall[pl.ds(lbase + r0, L)]
            lenA_ = jnp.full((L,), lv0[0], jnp.int32)
            lenB_ = jnp.full((L,), lv0[1], jnp.int32)
            len16 = jnp.where(b_of[g] > 0, lenB_, lenA_)
            m = l_of[g] < len16
            goff = ov[0] + jnp.minimum(lv0[0], jnp.int32(l0))
            plsc.store_compressed(ics[p].at[pl.ds(goff, L)], idx16, mask=m)
            plsc.store_compressed(dxs[p].at[pl.ds(goff, L)],
                                  row_off + jnp.int32(r0) + b_of[g], mask=m)
        return nc

    def s1_compute(t, p):
        valv = vals[p]
        lbase = t * CH
        for g in range(CH // L):
            lvi = lenall[pl.ds(lbase + g * L, L)]
            lvf = lvi.astype(jnp.float32)
            vsum = jnp.zeros((L,), jnp.float32)
            base_flat = jnp.int32(g * L * MAX_LEN) + iota16 * MAX_LEN
            for l in range(MAX_LEN):
                v = plsc.load_gather(valv, [base_flat + l])
                vsum = vsum + jnp.where(lvi > l, v, 0.0)
            s1s[p][pl.ds(g * L, L)] = vsum / (lvf * lvf)

    zeros = jnp.zeros((L,), jnp.float32)

    def zero_pooled(p):
        pltpu.sync_copy(zbuf, pls[p].at[pl.ds(pbase, CH + 1)])

    def fm(t, p):
        c = t % NCHUNK
        pltpu.sync_copy(pls[p].at[pl.ds(pbase, CH)], plloc)
        pool = plloc

        def rowbody(j, carry):
            for u in range(2):
                row = j * 2 + u
                jv = jnp.full((L,), row, jnp.int32)
                bs1 = plsc.load_gather(s1s[p], [jv])
                bs2 = bs1 * bs1
                brow = c * CH + row
                for q in range(NQ):
                    tq = pool[row, pl.ds(q * L, L)]
                    S1[brow, pl.ds(q * L, L)] = (
                        S1[brow, pl.ds(q * L, L)] + bs1 * tq)
                    S2[brow, pl.ds(q * L, L)] = (
                        S2[brow, pl.ds(q * L, L)] + bs2 * (tq * tq))
            return carry

        lax.fori_loop(0, L, rowbody, 0)

    def zinit(r, carry):
        for q in range(NQ):
            S1[r, pl.ds(q * L, L)] = zeros
            S2[r, pl.ds(q * L, L)] = zeros
        return carry

    lax.fori_loop(0, BSLAB, zinit, 0)

    def zbinit(r, carry):
        for q in range(NQ):
            zbuf[r, pl.ds(q * L, L)] = zeros
        return carry

    lax.fori_loop(0, CH + 1, zbinit, 0)

    def icinit(i, carry):
        base = i * L
        icA[pl.ds(base, L)] = iota16 + base
        icB[pl.ds(base, L)] = iota16 + base
        return carry

    lax.fori_loop(0, (GI + 128) // L, icinit, 0)

    for f in range(FIELD_SIZE):
        off = pl.multiple_of(f * BATCH + wid * BSLAB, 8)
        pltpu.async_copy(len_h.at[pl.ds(off, BSLAB)],
                         lenall.at[pl.ds(f * BSLAB, BSLAB)], semv0)
    for k in range((CH + L) // L):
        lenall[pl.ds(WR + k * L, L)] = jnp.zeros((L,), jnp.int32)
    for f in range(FIELD_SIZE):
        off = pl.multiple_of(f * BATCH + wid * BSLAB, 8)
        pltpu.make_async_copy(len_h.at[pl.ds(off, BSLAB)],
                              lenall.at[pl.ds(f * BSLAB, BSLAB)],
                              semv0).wait()

    pltpu.sync_copy(idx_src(0), idxs[0])
    nc0 = compact(0, 0)
    issue_gathers(0, nc0, jnp.bool_(True))
    issue_val(0, 0)
    issue_idx(1, 1)

    def step(t, p, nc_t):
        nxt = t + 1

        @pl.when(nxt < NT)
        def _():
            issue_val(nxt, 1 - p)

        @pl.when(t > 0)
        def _():
            wait_scatter(1 - p)
        @pl.when(t > 0)
        def _():
            fm(t - 1, 1 - p)

        wait_gathers(p, nc_t)
        wait_val(t, p)
        s1_compute(t, p)
        zero_pooled(p)
        issue_scatter(p)

        @pl.when(nxt < NT)
        def _():
            wait_idx(nxt, 1 - p)
        nc_n = compact(nxt, 1 - p)
        issue_gathers(1 - p, nc_n, nxt < NT)

        @pl.when(t + 2 < NT)
        def _():
            issue_idx(t + 2, p)
        return nc_n

    def pair(u, carry):
        nc_even, nc_odd = carry
        nc_odd = step(u * 2, 0, nc_even)
        nc_even = step(u * 2 + 1, 1, nc_odd)
        return (nc_even, nc_odd)

    lax.fori_loop(0, NT // 2, pair, (nc0, jnp.int32(0)))

    wait_scatter(1)
    fm(NT - 1, 1)

    def fin(r, carry):
        for q in range(NQ):
            a = S1[r, pl.ds(q * L, L)]
            b = S2[r, pl.ds(q * L, L)]
            S1[r, pl.ds(q * L, L)] = a * a - b
        return carry

    lax.fori_loop(0, BSLAB, fin, 0)

    pltpu.sync_copy(S1, out_h.at[pl.ds(wid * BSLAB, BSLAB)])


_mesh = plsc.VectorSubcoreMesh(core_axis_name="c", subcore_axis_name="s")

_sc_call = pl.kernel(
    _sc_body,
    out_type=jax.ShapeDtypeStruct((BATCH, EMB), jnp.float32),
    mesh=_mesh,
    scratch_types=[
        pltpu.VMEM((GI,), jnp.int32),
        pltpu.VMEM((GI,), jnp.int32),
        pltpu.VMEM((GI + 128,), jnp.int32),
        pltpu.VMEM((GI + 128,), jnp.int32),
        pltpu.VMEM((GI,), jnp.int32),
        pltpu.VMEM((GI,), jnp.int32),
        pltpu.VMEM((GI, EMB), jnp.float32),
        pltpu.VMEM((GI, EMB), jnp.float32),
        pltpu.VMEM((GI,), jnp.float32),
        pltpu.VMEM((GI,), jnp.float32),
        pltpu.VMEM((WR + CH + L,), jnp.int32),
        pltpu.VMEM((CH + L,), jnp.int32),
        pltpu.VMEM((CH + L,), jnp.int32),
        pltpu.VMEM((CH,), jnp.float32),
        pltpu.VMEM((CH,), jnp.float32),
        pltpu.VMEM_SHARED((NS * (CH + 1), EMB), jnp.float32),
        pltpu.VMEM_SHARED((NS * (CH + 1), EMB), jnp.float32),
        pltpu.VMEM((CH, EMB), jnp.float32),
        pltpu.VMEM((CH + 1, EMB), jnp.float32),
        pltpu.VMEM((BSLAB, EMB), jnp.float32),
        pltpu.VMEM((BSLAB, EMB), jnp.float32),
        pltpu.SemaphoreType.DMA,
        pltpu.SemaphoreType.DMA,
        pltpu.SemaphoreType.DMA,
        pltpu.SemaphoreType.DMA,
        pltpu.SemaphoreType.DMA,
        pltpu.SemaphoreType.DMA,
        pltpu.SemaphoreType.DMA,
        pltpu.SemaphoreType.DMA,
    ],
    compiler_params=pltpu.CompilerParams(needs_layout_passes=False,
                                         use_tc_tiling_on_sc=False),
)


@jax.jit
def kernel(feature_values, feature_idx, lengths, feature_embeddings):
    idxf = feature_idx.reshape(ROWS * MAX_LEN)
    valf = feature_values.reshape(ROWS * MAX_LEN)
    return _sc_call(idxf, valf, lengths, feature_embeddings)


# speedup vs baseline: 1.1924x; 1.1924x over previous
import jax
import jax.numpy as jnp
from jax import lax
from jax.experimental import pallas as pl
from jax.experimental.pallas import tpu as pltpu
from jax.experimental.pallas import tpu_sc as plsc

FEATURE_SIZE = 100000
FIELD_SIZE = 26
BATCH = 4096
EMB = 64
MAX_LEN = 20
ROWS = FIELD_SIZE * BATCH

NC, NS, L = 2, 16, 16
NW = NC * NS
BSLAB = BATCH // NW
CH = 32
NCHUNK = BSLAB // CH
NT = FIELD_SIZE * NCHUNK
GI = CH * MAX_LEN
NG = GI // 128
NQ = EMB // L
NGRP = GI // L
WR = FIELD_SIZE * BSLAB
TRASH = CH


def _sc_body(idx_h, val_h, len_h, tab_h, out_h,
             idxA, idxB, icA, icB, dxA, dxB, gA, gB, valA, valB,
             lenall, offA, offB, s1A, s1B, plA, plB, plloc, zbuf, S1, S2,
             semg0, semg1, semi0, semi1, semv0, semv1, sems0, sems1):
    sid = lax.axis_index("s")
    wid = sid * NC + lax.axis_index("c")
    pbase = sid * (CH + 1)

    idxs = (idxA, idxB)
    ics = (icA, icB)
    dxs = (dxA, dxB)
    gs = (gA, gB)
    vals = (valA, valB)
    offs = (offA, offB)
    s1s = (s1A, s1B)
    pls = (plA, plB)
    semg = (semg0, semg1)
    semi = (semi0, semi1)
    semv = (semv0, semv1)
    semsc = (sems0, sems1)

    iota16 = lax.iota(jnp.int32, L)
    l_of, b_of = [], []
    for g in range(NGRP):
        r0g = (g * L) // MAX_LEN
        th = (r0g + 1) * MAX_LEN - g * L
        bump = (iota16 >= th).astype(jnp.int32)
        b_of.append(bump)
        l_of.append(iota16 + jnp.int32(g * L - r0g * MAX_LEN)
                    - bump * MAX_LEN)

    def row0(t):
        f = t // NCHUNK
        c = t % NCHUNK
        return f * BATCH + wid * BSLAB + c * CH

    def idx_src(t):
        off = pl.multiple_of(row0(t) * MAX_LEN, 128)
        return idx_h.at[pl.ds(off, GI)]

    def val_src(t):
        off = pl.multiple_of(row0(t) * MAX_LEN, 128)
        return val_h.at[pl.ds(off, GI)]

    def issue_idx(t, p):
        pltpu.async_copy(idx_src(t), idxs[p], semi[p])

    def wait_idx(t, p):
        pltpu.make_async_copy(idx_src(t), idxs[p], semi[p]).wait()

    def issue_val(t, p):
        pltpu.async_copy(val_src(t), vals[p], semv[p])

    def wait_val(t, p):
        pltpu.make_async_copy(val_src(t), vals[p], semv[p]).wait()

    def issue_gathers(p, nc, live):
        for j in range(NG):
            @pl.when(jnp.logical_and(live, jnp.int32(j * 128) < nc))
            def _():
                pltpu.async_copy(
                    tab_h.at[ics[p].at[pl.ds(j * 128, 128)]],
                    gs[p].at[pl.ds(j * 128, 128)], semg[p])

    def wait_gathers(p, nc):
        for j in range(NG):
            @pl.when(jnp.int32(j * 128) < nc)
            def _():
                pltpu.make_async_copy(
                    tab_h.at[ics[p].at[pl.ds(j * 128, 128)]],
                    gs[p].at[pl.ds(j * 128, 128)], semg[p]).wait()

    def issue_scatter(p, nc):
        for j in range(NG):
            @pl.when(jnp.int32(j * 128) < nc)
            def _():
                pltpu.async_copy(
                    gs[p].at[pl.ds(j * 128, 128)],
                    pls[p].at[dxs[p].at[pl.ds(j * 128, 128)]],
                    semsc[p], add=True)

    def wait_scatter(p, nc):
        for j in range(NG):
            @pl.when(jnp.int32(j * 128) < nc)
            def _():
                pltpu.make_async_copy(
                    gs[p].at[pl.ds(j * 128, 128)],
                    pls[p].at[dxs[p].at[pl.ds(j * 128, 128)]],
                    semsc[p]).wait()

    def issue_zero(p):
        pltpu.async_copy(zbuf, pls[p].at[pl.ds(pbase, CH + 1)], semsc[p])

    def wait_zero(p):
        pltpu.make_async_copy(zbuf, pls[p].at[pl.ds(pbase, CH + 1)],
                              semsc[p]).wait()

    def compact(t, p):
        lbase = t * CH
        la = lenall[pl.ds(lbase, L)]
        lb = lenall[pl.ds(lbase + L, L)]
        offa = plsc.cumsum(la) - la
        offb = plsc.cumsum(lb) - lb
        tota = jnp.sum(la)
        offs[p][pl.ds(0, L)] = offa
        offs[p][pl.ds(L, L)] = offb + jnp.full((L,), tota, jnp.int32)
        nc = tota + jnp.sum(lb)
        trash16 = jnp.full((L,), pbase + CH, jnp.int32)
        row_off = jnp.full((L,), pbase, jnp.int32)
        for g in range(NGRP):
            dxs[p][pl.ds(g * L, L)] = trash16
        for g in range(NGRP):
            idx16 = idxs[p][pl.ds(g * L, L)]
            r0 = (g * L) // MAX_LEN
            l0 = (g * L) % MAX_LEN
            ov = offs[p][pl.ds(r0, L)]
            lv0 = lenall[pl.ds(lbase + r0, L)]
            lenA_ = jnp.full((L,), lv0[0], jnp.int32)
            lenB_ = jnp.full((L,), lv0[1], jnp.int32)
            len16 = jnp.where(b_of[g] > 0, lenB_, lenA_)
            m = l_of[g] < len16
            goff = ov[0] + jnp.minimum(lv0[0], jnp.int32(l0))
            plsc.store_compressed(ics[p].at[pl.ds(goff, L)], idx16, mask=m)
            plsc.store_compressed(dxs[p].at[pl.ds(goff, L)],
                                  row_off + jnp.int32(r0) + b_of[g], mask=m)
        return nc

    def s1_compute(t, p):
        valv = vals[p]
        lbase = t * CH
        for g in range(CH // L):
            lvi = lenall[pl.ds(lbase + g * L, L)]
            lvf = lvi.astype(jnp.float32)
            vsum = jnp.zeros((L,), jnp.float32)
            base_flat = jnp.int32(g * L * MAX_LEN) + iota16 * MAX_LEN
            for l in range(MAX_LEN):
                v = plsc.load_gather(valv, [base_flat + l])
                vsum = vsum + jnp.where(lvi > l, v, 0.0)
            s1s[p][pl.ds(g * L, L)] = vsum / (lvf * lvf)

    zeros = jnp.zeros((L,), jnp.float32)

    def fm(t, p):
        c = t % NCHUNK
        pltpu.sync_copy(pls[p].at[pl.ds(pbase, CH)], plloc)
        pool = plloc

        def rowbody(j, carry):
            for u in range(2):
                row = j * 2 + u
                jv = jnp.full((L,), row, jnp.int32)
                bs1 = plsc.load_gather(s1s[p], [jv])
                bs2 = bs1 * bs1
                brow = c * CH + row
                for q in range(NQ):
                    tq = pool[row, pl.ds(q * L, L)]
                    S1[brow, pl.ds(q * L, L)] = (
                        S1[brow, pl.ds(q * L, L)] + bs1 * tq)
                    S2[brow, pl.ds(q * L, L)] = (
                        S2[brow, pl.ds(q * L, L)] + bs2 * (tq * tq))
            return carry

        lax.fori_loop(0, L, rowbody, 0)

    def zinit(r, carry):
        for q in range(NQ):
            S1[r, pl.ds(q * L, L)] = zeros
            S2[r, pl.ds(q * L, L)] = zeros
        return carry

    lax.fori_loop(0, BSLAB, zinit, 0)

    def zbinit(r, carry):
        for q in range(NQ):
            zbuf[r, pl.ds(q * L, L)] = zeros
        return carry

    lax.fori_loop(0, CH + 1, zbinit, 0)

    def icinit(i, carry):
        base = i * L
        icA[pl.ds(base, L)] = iota16 + base
        icB[pl.ds(base, L)] = iota16 + base
        return carry

    lax.fori_loop(0, (GI + 128) // L, icinit, 0)

    for f in range(FIELD_SIZE):
        off = pl.multiple_of(f * BATCH + wid * BSLAB, 8)
        pltpu.async_copy(len_h.at[pl.ds(off, BSLAB)],
                         lenall.at[pl.ds(f * BSLAB, BSLAB)], semv0)
    for k in range((CH + L) // L):
        lenall[pl.ds(WR + k * L, L)] = jnp.zeros((L,), jnp.int32)
    for f in range(FIELD_SIZE):
        off = pl.multiple_of(f * BATCH + wid * BSLAB, 8)
        pltpu.make_async_copy(len_h.at[pl.ds(off, BSLAB)],
                              lenall.at[pl.ds(f * BSLAB, BSLAB)],
                              semv0).wait()

    pltpu.sync_copy(idx_src(0), idxs[0])
    nc0 = compact(0, 0)
    issue_gathers(0, nc0, jnp.bool_(True))
    issue_val(0, 0)
    issue_idx(1, 1)

    def step(t, p, nc_t, nc_prev):
        nxt = t + 1

        @pl.when(nxt < NT)
        def _():
            issue_val(nxt, 1 - p)

        issue_zero(p)

        @pl.when(t > 0)
        def _():
            wait_scatter(1 - p, nc_prev)
        @pl.when(t > 0)
        def _():
            fm(t - 1, 1 - p)

        wait_gathers(p, nc_t)
        wait_val(t, p)
        s1_compute(t, p)
        wait_zero(p)
        issue_scatter(p, nc_t)

        @pl.when(nxt < NT)
        def _():
            wait_idx(nxt, 1 - p)
        nc_n = compact(nxt, 1 - p)
        issue_gathers(1 - p, nc_n, nxt < NT)

        @pl.when(t + 2 < NT)
        def _():
            issue_idx(t + 2, p)
        return nc_n

    def pair(u, carry):
        nc_even, nc_odd = carry
        nc_odd_new = step(u * 2, 0, nc_even, nc_odd)
        nc_even_new = step(u * 2 + 1, 1, nc_odd_new, nc_even)
        return (nc_even_new, nc_odd_new)

    _, nc_last = lax.fori_loop(0, NT // 2, pair, (nc0, jnp.int32(0)))

    wait_scatter(1, nc_last)
    fm(NT - 1, 1)

    def fin(r, carry):
        for q in range(NQ):
            a = S1[r, pl.ds(q * L, L)]
            b = S2[r, pl.ds(q * L, L)]
            S1[r, pl.ds(q * L, L)] = a * a - b
        return carry

    lax.fori_loop(0, BSLAB, fin, 0)

    pltpu.sync_copy(S1, out_h.at[pl.ds(wid * BSLAB, BSLAB)])


_mesh = plsc.VectorSubcoreMesh(core_axis_name="c", subcore_axis_name="s")

_sc_call = pl.kernel(
    _sc_body,
    out_type=jax.ShapeDtypeStruct((BATCH, EMB), jnp.float32),
    mesh=_mesh,
    scratch_types=[
        pltpu.VMEM((GI,), jnp.int32),
        pltpu.VMEM((GI,), jnp.int32),
        pltpu.VMEM((GI + 128,), jnp.int32),
        pltpu.VMEM((GI + 128,), jnp.int32),
        pltpu.VMEM((GI,), jnp.int32),
        pltpu.VMEM((GI,), jnp.int32),
        pltpu.VMEM((GI, EMB), jnp.float32),
        pltpu.VMEM((GI, EMB), jnp.float32),
        pltpu.VMEM((GI,), jnp.float32),
        pltpu.VMEM((GI,), jnp.float32),
        pltpu.VMEM((WR + CH + L,), jnp.int32),
        pltpu.VMEM((CH + L,), jnp.int32),
        pltpu.VMEM((CH + L,), jnp.int32),
        pltpu.VMEM((CH,), jnp.float32),
        pltpu.VMEM((CH,), jnp.float32),
        pltpu.VMEM_SHARED((NS * (CH + 1), EMB), jnp.float32),
        pltpu.VMEM_SHARED((NS * (CH + 1), EMB), jnp.float32),
        pltpu.VMEM((CH, EMB), jnp.float32),
        pltpu.VMEM((CH + 1, EMB), jnp.float32),
        pltpu.VMEM((BSLAB, EMB), jnp.float32),
        pltpu.VMEM((BSLAB, EMB), jnp.float32),
        pltpu.SemaphoreType.DMA,
        pltpu.SemaphoreType.DMA,
        pltpu.SemaphoreType.DMA,
        pltpu.SemaphoreType.DMA,
        pltpu.SemaphoreType.DMA,
        pltpu.SemaphoreType.DMA,
        pltpu.SemaphoreType.DMA,
        pltpu.SemaphoreType.DMA,
    ],
    compiler_params=pltpu.CompilerParams(needs_layout_passes=False,
                                         use_tc_tiling_on_sc=False),
)


@jax.jit
def kernel(feature_values, feature_idx, lengths, feature_embeddings):
    idxf = feature_idx.reshape(ROWS * MAX_LEN)
    valf = feature_values.reshape(ROWS * MAX_LEN)
    return _sc_call(idxf, valf, lengths, feature_embeddings)
